# Initial kernel scaffold; baseline (speedup 1.0000x reference)
import jax
import jax.numpy as jnp
from jax.experimental import pallas as pl


def kernel(sentiment, table, W, b):
    raise NotImplementedError("write your pallas kernel here")


# speedup vs baseline: 3.2852x; 3.2852x over previous
import functools

import jax
import jax.numpy as jnp
from jax import lax
from jax.experimental import pallas as pl
from jax.experimental.pallas import tpu as pltpu
from jax.experimental.pallas import tpu_sc as plsc

NUM_ROWS = 1001
EMB = 64
OUT = 64
PAD_V = 1024

BATCH = 4096
HIST = 200
BTOT = BATCH * HIST

NC = 2
NS = 16
NW = NC * NS
CHUNK = 128
B_PER_W = BTOT // NW
NCH = B_PER_W // CHUNK


def _proj_body(tab_ref, w_ref, b_ref, out_ref):
    r = lax.broadcasted_iota(jnp.int32, (PAD_V, 1), 0)
    tab = jnp.where(r == 0, 0.0, tab_ref[...])
    acc = lax.dot_general(tab, w_ref[...], (((1,), (1,)), ((), ())),
                          preferred_element_type=jnp.float32)
    out_ref[...] = jnp.tanh(acc + b_ref[...])


def _project_table(tab_padded, W, b2):
    return pl.pallas_call(
        _proj_body,
        out_shape=jax.ShapeDtypeStruct((PAD_V, OUT), jnp.float32),
    )(tab_padded, W, b2)


def _gather_body(idx_hbm, p_hbm, out_hbm, idx_v, rows_v, sem):
    c = lax.axis_index("c")
    s = lax.axis_index("s")
    wid = s * NC + c
    pltpu.sync_copy(idx_hbm.at[wid], idx_v)
    base = wid * B_PER_W

    def body(g, carry):
        pltpu.async_copy(p_hbm.at[idx_v.at[g]], rows_v, sem).wait()
        pltpu.sync_copy(rows_v, out_hbm.at[pl.ds(base + g * CHUNK, CHUNK)])
        return carry

    lax.fori_loop(0, NCH, body, 0)


def _gather(idx3, P):
    mesh = plsc.VectorSubcoreMesh(core_axis_name="c", subcore_axis_name="s")
    return pl.kernel(
        _gather_body,
        mesh=mesh,
        compiler_params=pltpu.CompilerParams(use_tc_tiling_on_sc=False),
        out_type=jax.ShapeDtypeStruct((BTOT, OUT), jnp.float32),
        scratch_types=[
            pltpu.VMEM((NCH, CHUNK), jnp.int32),
            pltpu.VMEM((CHUNK, OUT), jnp.float32),
            pltpu.SemaphoreType.DMA,
        ],
    )(idx3, P)


def kernel(sentiment, table, W, b):
    idx3 = sentiment.reshape(NW, NCH, CHUNK)
    tab_p = jnp.zeros((PAD_V, EMB), table.dtype).at[:NUM_ROWS].set(table)
    P = _project_table(tab_p, W, b.reshape(1, OUT))
    out_flat = _gather(idx3, P)
    return out_flat.reshape(BATCH, HIST, OUT)


# speedup vs baseline: 3.3736x; 1.0269x over previous
import functools

import jax
import jax.numpy as jnp
from jax import lax
from jax.experimental import pallas as pl
from jax.experimental.pallas import tpu as pltpu
from jax.experimental.pallas import tpu_sc as plsc

NUM_ROWS = 1001
EMB = 64
OUT = 64
PAD_V = 1024

BATCH = 4096
HIST = 200
BTOT = BATCH * HIST

NC = 2
NS = 16
NW = NC * NS
CHUNK = 128
B_PER_W = BTOT // NW
NCH = B_PER_W // CHUNK


def _proj_body(tab_ref, w_ref, b_ref, out_ref):
    r = lax.broadcasted_iota(jnp.int32, (PAD_V, 1), 0)
    tab = jnp.where(r == 0, 0.0, tab_ref[...])
    acc = lax.dot_general(tab, w_ref[...], (((1,), (1,)), ((), ())),
                          preferred_element_type=jnp.float32)
    out_ref[...] = jnp.tanh(acc + b_ref[...])


def _project_table(tab_padded, W, b2):
    return pl.pallas_call(
        _proj_body,
        out_shape=jax.ShapeDtypeStruct((PAD_V, OUT), jnp.float32),
    )(tab_padded, W, b2)


NBUF = 8
PDIST = 4


def _gather_body(idx_hbm, p_hbm, out_hbm, idx_v, rows_v, gsem, osem):
    c = lax.axis_index("c")
    s = lax.axis_index("s")
    wid = s * NC + c
    pltpu.sync_copy(idx_hbm.at[wid], idx_v)
    base = wid * B_PER_W

    def fire_gather(g, b):
        pltpu.async_copy(p_hbm.at[idx_v.at[g]], rows_v.at[b], gsem.at[b])

    def wait_gather(g, b):
        pltpu.make_async_copy(p_hbm.at[idx_v.at[g]], rows_v.at[b],
                              gsem.at[b]).wait()

    def fire_store(g, b):
        pltpu.async_copy(rows_v.at[b],
                         out_hbm.at[pl.ds(base + g * CHUNK, CHUNK)],
                         osem.at[b])

    def wait_store(g, b):
        pltpu.make_async_copy(rows_v.at[b],
                              out_hbm.at[pl.ds(base + g * CHUNK, CHUNK)],
                              osem.at[b]).wait()

    for b in range(PDIST):
        fire_gather(b, b)

    def round_body(r, carry):
        t = r * NBUF
        for b in range(NBUF):
            g = t + b
            wait_gather(g, b)
            fire_store(g, b)
            ng = g + PDIST
            bn = (b + PDIST) % NBUF

            @pl.when(ng < NCH)
            def _():
                @pl.when(ng >= NBUF)
                def _():
                    wait_store(ng - NBUF, bn)
                fire_gather(ng, bn)
        return carry

    lax.fori_loop(0, NCH // NBUF, round_body, 0)

    for b in range(NBUF):
        g = NCH - NBUF + b
        wait_store(g, b)


def _gather(idx3, P):
    mesh = plsc.VectorSubcoreMesh(core_axis_name="c", subcore_axis_name="s")
    return pl.kernel(
        _gather_body,
        mesh=mesh,
        compiler_params=pltpu.CompilerParams(use_tc_tiling_on_sc=False),
        out_type=jax.ShapeDtypeStruct((BTOT, OUT), jnp.float32),
        scratch_types=[
            pltpu.VMEM((NCH, CHUNK), jnp.int32),
            pltpu.VMEM((NBUF, CHUNK, OUT), jnp.float32),
            pltpu.SemaphoreType.DMA((NBUF,)),
            pltpu.SemaphoreType.DMA((NBUF,)),
        ],
    )(idx3, P)


def kernel(sentiment, table, W, b):
    idx3 = sentiment.reshape(NW, NCH, CHUNK)
    tab_p = jnp.zeros((PAD_V, EMB), table.dtype).at[:NUM_ROWS].set(table)
    P = _project_table(tab_p, W, b.reshape(1, OUT))
    out_flat = _gather(idx3, P)
    return out_flat.reshape(BATCH, HIST, OUT)


# speedup vs baseline: 4.8456x; 1.4363x over previous
import functools

import jax
import jax.numpy as jnp
from jax import lax
from jax.experimental import pallas as pl
from jax.experimental.pallas import tpu as pltpu
from jax.experimental.pallas import tpu_sc as plsc

NUM_ROWS = 1001
EMB = 64
OUT = 64
PAD_V = 1024
PAD_O = 64

BATCH = 4096
HIST = 200
BTOT = BATCH * HIST

NC = 2
NS = 16
NW = NC * NS
CHUNK = 128
B_PER_W = BTOT // NW
NCH = B_PER_W // CHUNK

NBUF = 4
PDIST = 2


def _proj_body(tab_ref, w_ref, b_ref, out_ref):
    r = lax.broadcasted_iota(jnp.int32, (PAD_V, 1), 0)
    tab = jnp.where(r == 0, 0.0, tab_ref[...])
    acc = lax.dot_general(tab, w_ref[...], (((1,), (1,)), ((), ())),
                          preferred_element_type=jnp.float32)
    out_ref[...] = jnp.tanh(acc + b_ref[...])


def _project_table(tab_padded, Wp, b2):
    return pl.pallas_call(
        _proj_body,
        out_shape=jax.ShapeDtypeStruct((PAD_V, PAD_O), jnp.float32),
    )(tab_padded, Wp, b2)


def _gather_body(idx_hbm, p_hbm, out_hbm, idx_v, rows_v, p_sh, gsem, osem):
    c = lax.axis_index("c")
    s = lax.axis_index("s")
    wid = s * NC + c

    @pl.when(s == 0)
    def _():
        pltpu.sync_copy(p_hbm, p_sh)

    pltpu.sync_copy(idx_hbm.at[wid], idx_v)
    plsc.subcore_barrier()
    base = wid * B_PER_W

    def fire_gather(g, b):
        pltpu.async_copy(p_sh.at[idx_v.at[g]], rows_v.at[b], gsem.at[b])

    def wait_gather(g, b):
        pltpu.make_async_copy(p_sh.at[idx_v.at[g]], rows_v.at[b],
                              gsem.at[b]).wait()

    def fire_store(g, b):
        pltpu.async_copy(rows_v.at[b],
                         out_hbm.at[pl.ds(base + g * CHUNK, CHUNK)],
                         osem.at[b])

    def wait_store(g, b):
        pltpu.make_async_copy(rows_v.at[b],
                              out_hbm.at[pl.ds(base + g * CHUNK, CHUNK)],
                              osem.at[b]).wait()

    for b in range(PDIST):
        fire_gather(b, b)

    def round_body(r, carry):
        t = r * NBUF
        for b in range(NBUF):
            g = t + b
            wait_gather(g, b)
            fire_store(g, b)
            ng = g + PDIST
            bn = (b + PDIST) % NBUF

            @pl.when(ng < NCH)
            def _():
                @pl.when(ng >= NBUF)
                def _():
                    wait_store(ng - NBUF, bn)
                fire_gather(ng, bn)
        return carry

    lax.fori_loop(0, NCH // NBUF, round_body, 0)

    for b in range(NBUF):
        g = NCH - NBUF + b
        wait_store(g, b)


def _gather(idx3, P):
    mesh = plsc.VectorSubcoreMesh(core_axis_name="c", subcore_axis_name="s")
    return pl.kernel(
        _gather_body,
        mesh=mesh,
        compiler_params=pltpu.CompilerParams(use_tc_tiling_on_sc=False),
        out_type=jax.ShapeDtypeStruct((BTOT, OUT), jnp.float32),
        scratch_types=[
            pltpu.VMEM((NCH, CHUNK), jnp.int32),
            pltpu.VMEM((NBUF, CHUNK, PAD_O), jnp.float32),
            pltpu.MemorySpace.VMEM_SHARED((PAD_V, PAD_O), jnp.float32),
            pltpu.SemaphoreType.DMA((NBUF,)),
            pltpu.SemaphoreType.DMA((NBUF,)),
        ],
    )(idx3, P)


def kernel(sentiment, table, W, b):
    idx3 = sentiment.reshape(NW, NCH, CHUNK)
    tab_p = jnp.zeros((PAD_V, EMB), table.dtype).at[:NUM_ROWS].set(table)
    Wp = jnp.zeros((PAD_O, EMB), W.dtype).at[:OUT].set(W)
    b2 = jnp.zeros((1, PAD_O), b.dtype).at[0, :OUT].set(b)
    P = _project_table(tab_p, Wp, b2)
    out_flat = _gather(idx3, P)
    return out_flat.reshape(BATCH, HIST, OUT)
